# Initial kernel scaffold; baseline (speedup 1.0000x reference)
import jax
import jax.numpy as jnp
from jax.experimental import pallas as pl


def kernel(inputs, scale, targets_class):
    raise NotImplementedError("write your pallas kernel here")


# speedup vs baseline: 22.5461x; 22.5461x over previous
import functools

import jax
import jax.numpy as jnp
from jax.experimental import pallas as pl
from jax.experimental.pallas import tpu as pltpu

_K_PERCENT = 0.05
_BISECT_STEPS = 16


def _topk_mean_body(x_ref, out_ref, *, k):
    x = x_ref[0]
    kf = jnp.float32(k)
    lo0 = jnp.min(x) - 1.0
    hi0 = jnp.max(x)

    def step(_, carry):
        lo, hi = carry
        mid = 0.5 * (lo + hi)
        c = jnp.sum(jnp.where(x > mid, 1.0, 0.0))
        take = c >= kf
        return jnp.where(take, mid, lo), jnp.where(take, hi, mid)

    lo, hi = jax.lax.fori_loop(0, _BISECT_STEPS, step, (lo0, hi0))

    m_hi = x > hi
    m_lo = x > lo
    c_hi = jnp.sum(jnp.where(m_hi, 1.0, 0.0))
    c_lo = jnp.sum(jnp.where(m_lo, 1.0, 0.0))
    s_hi = jnp.sum(jnp.where(m_hi, x, 0.0))
    s_lo = jnp.sum(jnp.where(m_lo, x, 0.0))
    mu = (s_lo - s_hi) / jnp.maximum(c_lo - c_hi, 1.0)
    topk_sum = s_hi + (kf - c_hi) * mu
    out_ref[...] = jnp.full((1, 1, 128), topk_sum / kf, dtype=jnp.float32)


def _loss_body(p_ref, s_ref, t_ref, o_ref, *, nb, nc):
    z = p_ref[...]
    s = s_ref[0, 0]
    sp = jnp.maximum(s, 0.0) + jnp.log(1.0 + jnp.exp(-jnp.abs(s)))
    z = z * sp
    m = jnp.max(z, axis=1, keepdims=True)
    lse = m + jnp.log(jnp.sum(jnp.exp(z - m), axis=1, keepdims=True))
    lp = z - lse
    cols = jax.lax.broadcasted_iota(jnp.int32, (nb, nc), 1)
    sel = jnp.sum(jnp.where(cols == t_ref[...], lp, 0.0)) / nb
    o_ref[...] = jnp.full((8, 128), -sel, dtype=jnp.float32)


def kernel(inputs, scale, targets_class):
    B, C, H, W = inputs.shape
    hw = H * W
    k = max(1, int(hw * _K_PERCENT))
    rows = B * C
    assert hw % 128 == 0
    x = inputs.reshape(rows, hw // 128, 128)

    peaks = pl.pallas_call(
        functools.partial(_topk_mean_body, k=k),
        grid=(rows,),
        in_specs=[pl.BlockSpec((1, hw // 128, 128), lambda i: (i, 0, 0))],
        out_specs=pl.BlockSpec((1, 1, 128), lambda i: (i, 0, 0)),
        out_shape=jax.ShapeDtypeStruct((rows, 1, 128), jnp.float32),
        compiler_params=pltpu.CompilerParams(
            dimension_semantics=("arbitrary",),
        ),
    )(x)

    peak_logits = peaks[:, 0, 0].reshape(B, C)
    scale2d = scale.reshape(1, 1).astype(jnp.float32)
    tgt = targets_class.astype(jnp.int32).reshape(B, 1)

    loss = pl.pallas_call(
        functools.partial(_loss_body, nb=B, nc=C),
        in_specs=[
            pl.BlockSpec((B, C), lambda: (0, 0)),
            pl.BlockSpec((1, 1), lambda: (0, 0)),
            pl.BlockSpec((B, 1), lambda: (0, 0)),
        ],
        out_specs=pl.BlockSpec((8, 128), lambda: (0, 0)),
        out_shape=jax.ShapeDtypeStruct((8, 128), jnp.float32),
    )(peak_logits, scale2d, tgt)

    return loss[0, 0]
